# baseline (speedup 1.0000x reference)
import functools

import jax
import jax.numpy as jnp
from jax import lax
from jax.experimental import pallas as pl
from jax.experimental.pallas import tpu as pltpu
from jax.experimental.pallas import tpu_sc as plsc

N = 10000
E = 160000
D = 256
H = 128
NS = 16
NC = 2
EPT = E // NS
CH = 80
NCHUNK = EPT // CH
NG = 5
GC = NCHUNK // NG
NPAD = 10240
ROWS_PT = NPAD // NS
ZR = 40
L = 16
DR = NPAD // H


def _sc_body(x2, srcl, srch, dst2, zs, sums2, deg_out,
             sidx, didx, rows0, rows1, hist, sums_sh, deg_sh,
             sem0, sem1):
    ci = lax.axis_index("c")
    s = lax.axis_index("s")

    for r in range(ROWS_PT // ZR):
        pltpu.sync_copy(zs, sums_sh.at[pl.ds(s * ROWS_PT + r * ZR, ZR)])
    pltpu.sync_copy(zs, hist.at[pl.ds(0, ZR)])
    pltpu.sync_copy(zs, hist.at[pl.ds(ZR, ZR)])
    pltpu.sync_copy(zs.at[pl.ds(0, DR // NS)],
                    deg_sh.at[pl.ds(s * (DR // NS), DR // NS)])
    plsc.subcore_barrier()

    xh = x2
    onesv = jnp.full((L,), 1.0, jnp.float32)
    lanes = lax.iota(jnp.int32, L)

    def histo(j):
        @pl.when(ci == 0)
        def _():
            for k in range(CH // L):
                dv = didx[j, pl.ds(k * L, L)]
                r = jnp.right_shift(dv, 7)
                c = jnp.bitwise_and(dv, H - 1)
                plsc.addupdate_scatter(hist, [r, c], onesv)

    def group(g, carry):
        @pl.when(ci == 0)
        def _():
            pltpu.sync_copy(srcl.at[s, g], sidx)

        @pl.when(ci == 1)
        def _():
            pltpu.sync_copy(srch.at[s, g], sidx)

        pltpu.sync_copy(dst2.at[s, g], didx)

        pltpu.async_copy(xh.at[sidx.at[0]], rows0, sem0)

        def pair(p, carry2):
            j0 = 2 * p
            j1 = j0 + 1
            pltpu.make_async_copy(xh.at[sidx.at[j0]], rows0, sem0).wait()
            pltpu.async_copy(xh.at[sidx.at[j1]], rows1, sem1)
            pltpu.sync_copy(rows0, sums_sh.at[didx.at[j0]], add=True)
            histo(j0)
            pltpu.make_async_copy(xh.at[sidx.at[j1]], rows1, sem1).wait()
            pltpu.async_copy(xh.at[sidx.at[j0 + 2]], rows0, sem0)
            pltpu.sync_copy(rows1, sums_sh.at[didx.at[j1]], add=True)
            histo(j1)
            return carry2

        lax.fori_loop(0, (GC - 1) // 2, pair, 0)

        jl = GC - 1
        pltpu.make_async_copy(xh.at[sidx.at[jl]], rows0, sem0).wait()
        pltpu.sync_copy(rows0, sums_sh.at[didx.at[jl]], add=True)
        histo(jl)
        return carry

    lax.fori_loop(0, NG, group, 0)

    @pl.when(ci == 0)
    def _():
        for m in range(DR // L):
            pltpu.sync_copy(hist.at[pl.ds(m * L, L)],
                            deg_sh.at[lanes + m * L], add=True)

    plsc.subcore_barrier()

    pltpu.sync_copy(sums_sh.at[pl.ds(s * ROWS_PT, ROWS_PT)],
                    sums2.at[pl.ds(ci * NPAD + s * ROWS_PT, ROWS_PT)])

    @pl.when((ci == 0) & (s == 0))
    def _():
        pltpu.sync_copy(deg_sh, deg_out)


_sc_agg = functools.partial(
    pl.kernel,
    out_type=(
        jax.ShapeDtypeStruct((NC * NPAD, H), jnp.float32),
        jax.ShapeDtypeStruct((DR, H), jnp.float32),
    ),
    mesh=plsc.VectorSubcoreMesh(core_axis_name="c", subcore_axis_name="s"),
    compiler_params=pltpu.CompilerParams(needs_layout_passes=False),
    scratch_types=(
        pltpu.VMEM((GC, CH), jnp.int32),
        pltpu.VMEM((GC, CH), jnp.int32),
        pltpu.VMEM((CH, H), jnp.float32),
        pltpu.VMEM((CH, H), jnp.float32),
        pltpu.VMEM((DR, H), jnp.float32),
        pltpu.VMEM_SHARED((NPAD, H), jnp.float32),
        pltpu.VMEM_SHARED((DR, H), jnp.float32),
        pltpu.SemaphoreType.DMA,
        pltpu.SemaphoreType.DMA,
    ),
)(_sc_body)


BN = 1024


def _tc_self_body(x_ref, ws_ref, b_ref, o_ref):
    dn = (((1,), (1,)), ((), ()))
    o_ref[...] = lax.dot_general(x_ref[...], ws_ref[...], dn,
                                 preferred_element_type=jnp.float32) + b_ref[...]


_tc_self = pl.pallas_call(
    _tc_self_body,
    grid=((N + BN - 1) // BN,),
    in_specs=[
        pl.BlockSpec((BN, D), lambda i: (i, 0)),
        pl.BlockSpec((D, D), lambda i: (0, 0)),
        pl.BlockSpec((1, D), lambda i: (0, 0)),
    ],
    out_specs=pl.BlockSpec((BN, D), lambda i: (i, 0)),
    out_shape=jax.ShapeDtypeStruct((N, D), jnp.float32),
)


def _tc_body(hs_ref, slo_ref, shi_ref, deg_ref, wnlo_ref, wnhi_ref, o_ref):
    dd = deg_ref[...]
    deg = jnp.concatenate(
        [jnp.transpose(dd[t:t + 1, :]) for t in range(BN // H)], axis=0)
    inv = jnp.where(deg > 0, 1.0 / jnp.maximum(deg, 1.0), 0.0)
    dn = (((1,), (1,)), ((), ()))
    acc = hs_ref[...]
    acc = acc + lax.dot_general(slo_ref[...] * inv, wnlo_ref[...], dn,
                                preferred_element_type=jnp.float32)
    acc = acc + lax.dot_general(shi_ref[...] * inv, wnhi_ref[...], dn,
                                preferred_element_type=jnp.float32)
    o_ref[...] = acc


_tc_combine = pl.pallas_call(
    _tc_body,
    grid=((N + BN - 1) // BN,),
    in_specs=[
        pl.BlockSpec((BN, D), lambda i: (i, 0)),
        pl.BlockSpec((BN, H), lambda i: (i, 0)),
        pl.BlockSpec((BN, H), lambda i: (i + NPAD // BN, 0)),
        pl.BlockSpec((8, H), lambda i: (i, 0)),
        pl.BlockSpec((D, H), lambda i: (0, 0)),
        pl.BlockSpec((D, H), lambda i: (0, 1)),
    ],
    out_specs=pl.BlockSpec((BN, D), lambda i: (i, 0)),
    out_shape=jax.ShapeDtypeStruct((N, D), jnp.float32),
)


@jax.jit
def kernel(x, edge_index, W_self, b_self, W_neigh, bias):
    x2 = x.reshape(N, NC, H).transpose(1, 0, 2).reshape(NC * N, H)
    srcl = edge_index[0].reshape(NS, NG, GC, CH)
    srch = srcl + N
    dst2 = edge_index[1].reshape(NS, NG, GC, CH)
    zs = jnp.zeros((ZR, H), jnp.float32)

    sums2, deg = _sc_agg(x2, srcl, srch, dst2, zs)

    bias_all = (b_self + bias).reshape(1, D)
    h_self = _tc_self(x, W_self, bias_all)
    return _tc_combine(h_self, sums2, sums2, deg, W_neigh, W_neigh)


# speedup vs baseline: 1.0250x; 1.0250x over previous
import functools

import jax
import jax.numpy as jnp
from jax import lax
from jax.experimental import pallas as pl
from jax.experimental.pallas import tpu as pltpu
from jax.experimental.pallas import tpu_sc as plsc

N = 10000
E = 160000
D = 256
H = 128
NS = 16
NC = 2
EPT = E // NS
CH = 80
CHD = CH // 2
NCHUNK = EPT // CH
NG = 5
GC = NCHUNK // NG
NPAD = 10240
ROWS_PT = NPAD // NS
ZR = 40
L = 16
DR = NPAD // H


def _sc_body(x2, srcl, srch, dst2, zs, sums2, deg_out,
             sidx, didx, rows0, rows1, hist, sums_sh, deg_sh,
             sem0, sem1, sema, semb):
    ci = lax.axis_index("c")
    s = lax.axis_index("s")

    for r in range(ROWS_PT // ZR):
        pltpu.sync_copy(zs, sums_sh.at[pl.ds(s * ROWS_PT + r * ZR, ZR)])
    pltpu.sync_copy(zs, hist.at[pl.ds(0, ZR)])
    pltpu.sync_copy(zs, hist.at[pl.ds(ZR, ZR)])
    pltpu.sync_copy(zs.at[pl.ds(0, DR // NS)],
                    deg_sh.at[pl.ds(s * (DR // NS), DR // NS)])
    plsc.subcore_barrier()

    xh = x2
    onesv = jnp.full((L,), 1.0, jnp.float32)
    lanes2 = lax.iota(jnp.int32, L)
    lanes = lax.iota(jnp.int32, L)

    def histo(j):
        @pl.when(ci == 0)
        def _():
            for jj in (2 * j, 2 * j + 1):
                for k, msk in ((0, None), (16, None),
                               (24, lanes >= jnp.int32(8))):
                    dv = didx[jj, pl.ds(k, L)]
                    r = jnp.right_shift(dv, 7)
                    c = jnp.bitwise_and(dv, H - 1)
                    plsc.addupdate_scatter(hist, [r, c], onesv, mask=msk)

    def group(g, carry):
        @pl.when(ci == 0)
        def _():
            pltpu.sync_copy(srcl.at[s, g], sidx)

        @pl.when(ci == 1)
        def _():
            pltpu.sync_copy(srch.at[s, g], sidx)

        pltpu.sync_copy(dst2.at[s, g], didx)

        pltpu.async_copy(xh.at[sidx.at[0]], rows0, sem0)

        def pair(p, carry2):
            j0 = 2 * p
            j1 = j0 + 1
            pltpu.make_async_copy(xh.at[sidx.at[j0]], rows0, sem0).wait()
            pltpu.async_copy(xh.at[sidx.at[j1]], rows1, sem1)
            pltpu.async_copy(rows0.at[pl.ds(0, CHD)],
                             sums_sh.at[didx.at[2 * j0]], sema, add=True)
            pltpu.async_copy(rows0.at[pl.ds(CHD, CHD)],
                             sums_sh.at[didx.at[2 * j0 + 1]], semb, add=True)
            histo(j0)
            pltpu.make_async_copy(rows0.at[pl.ds(0, CHD)],
                                  sums_sh.at[didx.at[2 * j0]], sema).wait()
            pltpu.make_async_copy(rows0.at[pl.ds(CHD, CHD)],
                                  sums_sh.at[didx.at[2 * j0 + 1]], semb).wait()
            histo(j1)
            pltpu.make_async_copy(xh.at[sidx.at[j1]], rows1, sem1).wait()
            pltpu.async_copy(xh.at[sidx.at[j0 + 2]], rows0, sem0)
            pltpu.async_copy(rows1.at[pl.ds(0, CHD)],
                             sums_sh.at[didx.at[2 * j1]], sema, add=True)
            pltpu.async_copy(rows1.at[pl.ds(CHD, CHD)],
                             sums_sh.at[didx.at[2 * j1 + 1]], semb, add=True)
            pltpu.make_async_copy(rows1.at[pl.ds(0, CHD)],
                                  sums_sh.at[didx.at[2 * j1]], sema).wait()
            pltpu.make_async_copy(rows1.at[pl.ds(CHD, CHD)],
                                  sums_sh.at[didx.at[2 * j1 + 1]], semb).wait()
            return carry2

        lax.fori_loop(0, (GC - 1) // 2, pair, 0)

        jl = GC - 1
        pltpu.make_async_copy(xh.at[sidx.at[jl]], rows0, sem0).wait()
        pltpu.async_copy(rows0.at[pl.ds(0, CHD)],
                         sums_sh.at[didx.at[2 * jl]], sema, add=True)
        pltpu.async_copy(rows0.at[pl.ds(CHD, CHD)],
                         sums_sh.at[didx.at[2 * jl + 1]], semb, add=True)
        histo(jl)
        pltpu.make_async_copy(rows0.at[pl.ds(0, CHD)],
                              sums_sh.at[didx.at[2 * jl]], sema).wait()
        pltpu.make_async_copy(rows0.at[pl.ds(CHD, CHD)],
                              sums_sh.at[didx.at[2 * jl + 1]], semb).wait()
        return carry

    lax.fori_loop(0, NG, group, 0)

    @pl.when(ci == 0)
    def _():
        for m in range(DR // L):
            pltpu.sync_copy(hist.at[pl.ds(m * L, L)],
                            deg_sh.at[lanes + m * L], add=True)

    plsc.subcore_barrier()

    pltpu.sync_copy(sums_sh.at[pl.ds(s * ROWS_PT, ROWS_PT)],
                    sums2.at[pl.ds(ci * NPAD + s * ROWS_PT, ROWS_PT)])

    @pl.when((ci == 0) & (s == 0))
    def _():
        pltpu.sync_copy(deg_sh, deg_out)


_sc_agg = functools.partial(
    pl.kernel,
    out_type=(
        jax.ShapeDtypeStruct((NC * NPAD, H), jnp.float32),
        jax.ShapeDtypeStruct((DR, H), jnp.float32),
    ),
    mesh=plsc.VectorSubcoreMesh(core_axis_name="c", subcore_axis_name="s"),
    compiler_params=pltpu.CompilerParams(needs_layout_passes=False),
    scratch_types=(
        pltpu.VMEM((GC, CH), jnp.int32),
        pltpu.VMEM((2 * GC, CHD), jnp.int32),
        pltpu.VMEM((CH, H), jnp.float32),
        pltpu.VMEM((CH, H), jnp.float32),
        pltpu.VMEM((DR, H), jnp.float32),
        pltpu.VMEM_SHARED((NPAD, H), jnp.float32),
        pltpu.VMEM_SHARED((DR, H), jnp.float32),
        pltpu.SemaphoreType.DMA,
        pltpu.SemaphoreType.DMA,
        pltpu.SemaphoreType.DMA,
        pltpu.SemaphoreType.DMA,
    ),
)(_sc_body)


BN = 1024


def _tc_body(x_ref, slo_ref, shi_ref, deg_ref, ws_ref, wnlo_ref, wnhi_ref,
             b_ref, o_ref):
    dd = deg_ref[...]
    deg = jnp.concatenate(
        [jnp.transpose(dd[t:t + 1, :]) for t in range(BN // H)], axis=0)
    inv = jnp.where(deg > 0, 1.0 / jnp.maximum(deg, 1.0), 0.0)
    dn = (((1,), (1,)), ((), ()))
    acc = lax.dot_general(x_ref[...], ws_ref[...], dn,
                          preferred_element_type=jnp.float32)
    acc = acc + lax.dot_general(slo_ref[...] * inv, wnlo_ref[...], dn,
                                preferred_element_type=jnp.float32)
    acc = acc + lax.dot_general(shi_ref[...] * inv, wnhi_ref[...], dn,
                                preferred_element_type=jnp.float32)
    o_ref[...] = acc + b_ref[...]


_tc_combine = pl.pallas_call(
    _tc_body,
    grid=((N + BN - 1) // BN,),
    in_specs=[
        pl.BlockSpec((BN, D), lambda i: (i, 0)),
        pl.BlockSpec((BN, H), lambda i: (i, 0)),
        pl.BlockSpec((BN, H), lambda i: (i + NPAD // BN, 0)),
        pl.BlockSpec((8, H), lambda i: (i, 0)),
        pl.BlockSpec((D, D), lambda i: (0, 0)),
        pl.BlockSpec((D, H), lambda i: (0, 0)),
        pl.BlockSpec((D, H), lambda i: (0, 1)),
        pl.BlockSpec((1, D), lambda i: (0, 0)),
    ],
    out_specs=pl.BlockSpec((BN, D), lambda i: (i, 0)),
    out_shape=jax.ShapeDtypeStruct((N, D), jnp.float32),
)


@jax.jit
def kernel(x, edge_index, W_self, b_self, W_neigh, bias):
    x2 = x.reshape(N, NC, H).transpose(1, 0, 2).reshape(NC * N, H)
    srcl = edge_index[0].reshape(NS, NG, GC, CH)
    srch = srcl + N
    dst2 = edge_index[1].reshape(NS, NG, 2 * GC, CHD)
    zs = jnp.zeros((ZR, H), jnp.float32)

    sums2, deg = _sc_agg(x2, srcl, srch, dst2, zs)

    bias_all = (b_self + bias).reshape(1, D)
    return _tc_combine(x, sums2, sums2, deg, W_self, W_neigh, W_neigh,
                       bias_all)


# speedup vs baseline: 1.0394x; 1.0141x over previous
import functools

import jax
import jax.numpy as jnp
from jax import lax
from jax.experimental import pallas as pl
from jax.experimental.pallas import tpu as pltpu
from jax.experimental.pallas import tpu_sc as plsc

N = 10000
E = 160000
D = 256
H = 128
NS = 16
NC = 2
EPT = E // NS
CH = 80
NCHUNK = EPT // CH
NG = 5
GC = NCHUNK // NG
NPAD = 10240
ROWS_PT = NPAD // NS
ZR = 40
L = 16
DR = NPAD // H


def _sc_body(x2, srcl, srch, dst2, zs, sums2, deg_out,
             sidx, didx, rows0, rows1, hist, sums_sh, deg_sh,
             sem0, sem1):
    ci = lax.axis_index("c")
    s = lax.axis_index("s")

    for r in range(ROWS_PT // ZR):
        pltpu.sync_copy(zs, sums_sh.at[pl.ds(s * ROWS_PT + r * ZR, ZR)])
    pltpu.sync_copy(zs, hist.at[pl.ds(0, ZR)])
    pltpu.sync_copy(zs, hist.at[pl.ds(ZR, ZR)])
    pltpu.sync_copy(zs.at[pl.ds(0, DR // NS)],
                    deg_sh.at[pl.ds(s * (DR // NS), DR // NS)])
    plsc.subcore_barrier()

    xh = x2
    onesv = jnp.full((L,), 1.0, jnp.float32)
    lanes = lax.iota(jnp.int32, L)

    def histo(j):
        @pl.when(ci == 0)
        def _():
            for k in range(CH // L):
                dv = didx[j, pl.ds(k * L, L)]
                r = jnp.right_shift(dv, 7)
                c = jnp.bitwise_and(dv, H - 1)
                plsc.addupdate_scatter(hist, [r, c], onesv)

    def group(g, carry):
        @pl.when(ci == 0)
        def _():
            pltpu.sync_copy(srcl.at[s, g], sidx)

        @pl.when(ci == 1)
        def _():
            pltpu.sync_copy(srch.at[s, g], sidx)

        pltpu.sync_copy(dst2.at[s, g], didx)

        pltpu.async_copy(xh.at[sidx.at[0]], rows0, sem0)

        def pair(p, carry2):
            j0 = 2 * p
            j1 = j0 + 1
            pltpu.make_async_copy(xh.at[sidx.at[j0]], rows0, sem0).wait()
            pltpu.async_copy(xh.at[sidx.at[j1]], rows1, sem1)
            pltpu.sync_copy(rows0, sums_sh.at[didx.at[j0]], add=True)
            histo(j0)
            pltpu.make_async_copy(xh.at[sidx.at[j1]], rows1, sem1).wait()
            pltpu.async_copy(xh.at[sidx.at[j0 + 2]], rows0, sem0)
            pltpu.sync_copy(rows1, sums_sh.at[didx.at[j1]], add=True)
            histo(j1)
            return carry2

        lax.fori_loop(0, (GC - 1) // 2, pair, 0)

        jl = GC - 1
        pltpu.make_async_copy(xh.at[sidx.at[jl]], rows0, sem0).wait()
        pltpu.sync_copy(rows0, sums_sh.at[didx.at[jl]], add=True)
        histo(jl)
        return carry

    lax.fori_loop(0, NG, group, 0)

    @pl.when(ci == 0)
    def _():
        for m in range(DR // L):
            pltpu.sync_copy(hist.at[pl.ds(m * L, L)],
                            deg_sh.at[lanes + m * L], add=True)

    plsc.subcore_barrier()

    pltpu.sync_copy(sums_sh.at[pl.ds(s * ROWS_PT, ROWS_PT)],
                    sums2.at[pl.ds(ci * NPAD + s * ROWS_PT, ROWS_PT)])

    @pl.when((ci == 0) & (s == 0))
    def _():
        pltpu.sync_copy(deg_sh, deg_out)


_sc_agg = functools.partial(
    pl.kernel,
    out_type=(
        jax.ShapeDtypeStruct((NC * NPAD, H), jnp.float32),
        jax.ShapeDtypeStruct((DR, H), jnp.float32),
    ),
    mesh=plsc.VectorSubcoreMesh(core_axis_name="c", subcore_axis_name="s"),
    compiler_params=pltpu.CompilerParams(needs_layout_passes=False),
    scratch_types=(
        pltpu.VMEM((GC, CH), jnp.int32),
        pltpu.VMEM((GC, CH), jnp.int32),
        pltpu.VMEM((CH, H), jnp.float32),
        pltpu.VMEM((CH, H), jnp.float32),
        pltpu.VMEM((DR, H), jnp.float32),
        pltpu.VMEM_SHARED((NPAD, H), jnp.float32),
        pltpu.VMEM_SHARED((DR, H), jnp.float32),
        pltpu.SemaphoreType.DMA,
        pltpu.SemaphoreType.DMA,
    ),
)(_sc_body)


BN = 2048


def _tc_body(x_ref, slo_ref, shi_ref, deg_ref, ws_ref, wnlo_ref, wnhi_ref,
             b_ref, o_ref):
    dd = deg_ref[...]
    deg = jnp.concatenate(
        [jnp.transpose(dd[t:t + 1, :]) for t in range(BN // H)], axis=0)
    inv = jnp.where(deg > 0, 1.0 / jnp.maximum(deg, 1.0), 0.0)
    dn = (((1,), (1,)), ((), ()))
    acc = lax.dot_general(x_ref[...], ws_ref[...], dn,
                          preferred_element_type=jnp.float32)
    acc = acc + lax.dot_general(slo_ref[...] * inv, wnlo_ref[...], dn,
                                preferred_element_type=jnp.float32)
    acc = acc + lax.dot_general(shi_ref[...] * inv, wnhi_ref[...], dn,
                                preferred_element_type=jnp.float32)
    o_ref[...] = acc + b_ref[...]


_tc_combine = pl.pallas_call(
    _tc_body,
    grid=((N + BN - 1) // BN,),
    in_specs=[
        pl.BlockSpec((BN, D), lambda i: (i, 0)),
        pl.BlockSpec((BN, H), lambda i: (i, 0)),
        pl.BlockSpec((BN, H), lambda i: (i + NPAD // BN, 0)),
        pl.BlockSpec((BN // H, H), lambda i: (i, 0)),
        pl.BlockSpec((D, D), lambda i: (0, 0)),
        pl.BlockSpec((D, H), lambda i: (0, 0)),
        pl.BlockSpec((D, H), lambda i: (0, 1)),
        pl.BlockSpec((1, D), lambda i: (0, 0)),
    ],
    out_specs=pl.BlockSpec((BN, D), lambda i: (i, 0)),
    out_shape=jax.ShapeDtypeStruct((N, D), jnp.float32),
)


@jax.jit
def kernel(x, edge_index, W_self, b_self, W_neigh, bias):
    x2 = x.reshape(N, NC, H).transpose(1, 0, 2).reshape(NC * N, H)
    srcl = edge_index[0].reshape(NS, NG, GC, CH)
    srch = srcl + N
    dst2 = edge_index[1].reshape(NS, NG, GC, CH)
    zs = jnp.zeros((ZR, H), jnp.float32)

    sums2, deg = _sc_agg(x2, srcl, srch, dst2, zs)

    bias_all = (b_self + bias).reshape(1, D)
    return _tc_combine(x, sums2, sums2, deg, W_self, W_neigh, W_neigh,
                       bias_all)
